# Initial kernel scaffold; baseline (speedup 1.0000x reference)
import functools

import jax
import jax.numpy as jnp
import numpy as np
from jax import lax
from jax.experimental import pallas as pl
from jax.experimental.pallas import tpu as pltpu
from jax.experimental.pallas import tpu_sc as plsc

D = 256
NH = 8
NL = 4
NP = 4
HD = D // NH
NTAP = 4
_SHAPES = [(128, 128), (64, 64), (32, 32), (16, 16)]
LQ = sum(h * w for h, w in _SHAPES)
B = 2
BLK = 640
NBLK = LQ // BLK
E = NH * NL * NP * NTAP
ROWS = B * LQ * NH

_lane = np.arange(128)
_lane_l = (_lane // 4) % 4
_lane_h = _lane // 16
_W_l = np.array([_SHAPES[l][1] for l in _lane_l], np.float32)
_H_l = np.array([_SHAPES[l][0] for l in _lane_l], np.float32)
_lsi_np = np.cumsum([0] + [h * w for h, w in _SHAPES])[:-1]
_LSI_H = (_lsi_np[_lane_l] * NH + _lane_h).astype(np.int32)
_Sx = np.zeros((NL, 128), np.float32)
_Sx[_lane_l, _lane] = _W_l
_Sy = np.zeros((NL, 128), np.float32)
_Sy[_lane_l, _lane] = _H_l
_G = (_lane_h[:, None] == _lane_h[None, :]).astype(np.float32)


def _stage1_body(q_ref, v_ref, wv_ref, bv_ref, wa_ref, ba_ref, wox_ref,
                 woy_ref, bx_ref, by_ref, sx_ref, sy_ref, wl_ref, hl_ref,
                 lsih_ref, g_ref, rpx_ref, rpy_ref,
                 val_ref, idx_ref, w_ref):
    b = pl.program_id(0)
    q = q_ref[0]
    v = v_ref[0]
    val_ref[0] = jnp.dot(v, wv_ref[...], preferred_element_type=jnp.float32) + bv_ref[...]
    logits = jnp.dot(q, wa_ref[...], preferred_element_type=jnp.float32) + ba_ref[...]
    m = jnp.max(logits, axis=1, keepdims=True)
    e = jnp.exp(logits - m)
    den = jnp.dot(e, g_ref[...], preferred_element_type=jnp.float32)
    att = e / den
    x = (jnp.dot(q, wox_ref[...], preferred_element_type=jnp.float32)
         + jnp.dot(rpx_ref[0], sx_ref[...], preferred_element_type=jnp.float32)
         + bx_ref[...])
    y = (jnp.dot(q, woy_ref[...], preferred_element_type=jnp.float32)
         + jnp.dot(rpy_ref[0], sy_ref[...], preferred_element_type=jnp.float32)
         + by_ref[...])
    wl = wl_ref[...]
    hl = hl_ref[...]
    x0 = jnp.floor(x)
    y0 = jnp.floor(y)
    fx1 = x - x0
    fx0 = 1.0 - fx1
    fy1 = y - y0
    fy0 = 1.0 - fy1
    base_b = b * (LQ * NH)
    lsih = lsih_ref[...]
    for t, (dx, dy) in enumerate(((0, 0), (1, 0), (0, 1), (1, 1))):
        xt = x0 + float(dx)
        yt = y0 + float(dy)
        valid = ((xt >= 0.0) & (xt <= wl - 1.0)
                 & (yt >= 0.0) & (yt <= hl - 1.0))
        xc = jnp.clip(xt, 0.0, wl - 1.0).astype(jnp.int32)
        yc = jnp.clip(yt, 0.0, hl - 1.0).astype(jnp.int32)
        spatial = yc * wl.astype(jnp.int32) + xc
        idx_ref[0, :, t * 128:(t + 1) * 128] = base_b + lsih + spatial * NH
        wt = (fx0 if dx == 0 else fx1) * (fy0 if dy == 0 else fy1)
        w_ref[0, :, t * 128:(t + 1) * 128] = att * wt * valid.astype(jnp.float32)


def _stage1(query, value, rpx, rpy, W_val, b_val, W_attn, b_attn,
            W_offx, W_offy, bx, by):
    consts = [
        jnp.asarray(_Sx), jnp.asarray(_Sy),
        jnp.asarray(_W_l).reshape(1, 128), jnp.asarray(_H_l).reshape(1, 128),
        jnp.asarray(_LSI_H).reshape(1, 128), jnp.asarray(_G),
    ]

    def whole(shape):
        return pl.BlockSpec(shape, lambda b, i: tuple(0 for _ in shape))

    return pl.pallas_call(
        _stage1_body,
        grid=(B, NBLK),
        in_specs=[
            pl.BlockSpec((1, BLK, D), lambda b, i: (b, i, 0)),
            pl.BlockSpec((1, BLK, D), lambda b, i: (b, i, 0)),
            whole((D, D)), whole((1, D)),
            whole((D, 128)), whole((1, 128)),
            whole((D, 128)), whole((D, 128)),
            whole((1, 128)), whole((1, 128)),
            whole((NL, 128)), whole((NL, 128)),
            whole((1, 128)), whole((1, 128)),
            whole((1, 128)), whole((128, 128)),
            pl.BlockSpec((1, BLK, NL), lambda b, i: (b, i, 0)),
            pl.BlockSpec((1, BLK, NL), lambda b, i: (b, i, 0)),
        ],
        out_specs=[
            pl.BlockSpec((1, BLK, D), lambda b, i: (b, i, 0)),
            pl.BlockSpec((1, BLK, E), lambda b, i: (b, i, 0)),
            pl.BlockSpec((1, BLK, E), lambda b, i: (b, i, 0)),
        ],
        out_shape=[
            jax.ShapeDtypeStruct((B, LQ, D), jnp.float32),
            jax.ShapeDtypeStruct((B, LQ, E), jnp.int32),
            jax.ShapeDtypeStruct((B, LQ, E), jnp.float32),
        ],
        compiler_params=pltpu.CompilerParams(
            dimension_semantics=("parallel", "parallel")),
    )(query, value, W_val, b_val, W_attn, b_attn, W_offx, W_offy, bx, by,
      *consts, rpx, rpy)


QC = 4
NQ = B * LQ
NGATH = E // 128


def _sc_body(tab_hbm, idx_hbm, w_hbm, out_hbm, idx_v, w_v, rows_v, out_v, sem):
    nc = 2
    wid = lax.axis_index("s") * nc + lax.axis_index("c")
    per_tile = NQ // 32
    iters = per_tile // QC
    qbase = wid * per_tile

    def it_body(i, carry):
        q0 = qbase + i * QC
        pltpu.sync_copy(idx_hbm.at[pl.ds(q0 * NGATH, QC * NGATH)], idx_v)
        pltpu.sync_copy(w_hbm.at[pl.ds(q0, QC)], w_v)
        handles = []
        for j in range(QC * NGATH):
            handles.append(pltpu.async_copy(
                tab_hbm.at[idx_v.at[j]],
                rows_v.at[pl.ds(j * 128, 128)], sem))
        for h in handles:
            h.wait()

        def qh_body(qh, c2):
            q = lax.shift_right_logical(qh, 3)
            hh = jnp.bitwise_and(qh, 7)

            def t_body(t, accs):
                a0, a1 = accs
                base = t * 128 + hh * 16
                for k in range(16):
                    ee = base + k
                    wv = w_v[q, ee]
                    r = q * E + ee
                    a0 = a0 + wv * rows_v[r, 0:16]
                    a1 = a1 + wv * rows_v[r, 16:32]
                return (a0, a1)

            z = jnp.zeros((16,), jnp.float32)
            a0, a1 = lax.fori_loop(0, NTAP, t_body, (z, z))
            out_v[qh, 0:16] = a0
            out_v[qh, 16:32] = a1
            return c2

        lax.fori_loop(0, QC * NH, qh_body, 0)
        pltpu.sync_copy(out_v, out_hbm.at[pl.ds(q0 * NH, QC * NH)])
        return carry

    lax.fori_loop(0, iters, it_body, 0)


def _combine_sc(tab, idx4, w2):
    mesh = plsc.VectorSubcoreMesh(core_axis_name="c", subcore_axis_name="s")
    run = functools.partial(
        pl.kernel,
        mesh=mesh,
        out_type=jax.ShapeDtypeStruct((ROWS, HD), jnp.float32),
        scratch_types=[
            pltpu.VMEM((QC * NGATH, 128), jnp.int32),
            pltpu.VMEM((QC, E), jnp.float32),
            pltpu.VMEM((QC * E, HD), jnp.float32),
            pltpu.VMEM((QC * NH, HD), jnp.float32),
            pltpu.SemaphoreType.DMA,
        ],
    )(_sc_body)
    return run(tab, idx4, w2)


def _stage2_body(x_ref, w_ref, b_ref, o_ref):
    o_ref[0] = (jnp.dot(x_ref[0], w_ref[...], preferred_element_type=jnp.float32)
                + b_ref[...])


def _stage2(x, W_out, b_out):
    return pl.pallas_call(
        _stage2_body,
        grid=(B, NBLK),
        in_specs=[
            pl.BlockSpec((1, BLK, D), lambda b, i: (b, i, 0)),
            pl.BlockSpec((D, D), lambda b, i: (0, 0)),
            pl.BlockSpec((1, D), lambda b, i: (0, 0)),
        ],
        out_specs=pl.BlockSpec((1, BLK, D), lambda b, i: (b, i, 0)),
        out_shape=jax.ShapeDtypeStruct((B, LQ, D), jnp.float32),
        compiler_params=pltpu.CompilerParams(
            dimension_semantics=("parallel", "parallel")),
    )(x, W_out, b_out)


def kernel(query, reference_points, value, spatial_shapes, level_start_index,
           W_off, b_off, W_attn, b_attn, W_val, b_val, W_out, b_out):
    rpx = reference_points[..., 0]
    rpy = reference_points[..., 1]
    W_offx = W_off[:, 0::2]
    W_offy = W_off[:, 1::2]
    bx = (b_off[0::2] - 0.5).reshape(1, 128)
    by = (b_off[1::2] - 0.5).reshape(1, 128)
    val, idxs, ws = _stage1(
        query, value, rpx, rpy, W_val, b_val.reshape(1, D),
        W_attn, b_attn.reshape(1, 128), W_offx, W_offy, bx, by)
    tab = val.reshape(ROWS, HD)
    idx4 = idxs.reshape(NQ * NGATH, 128)
    w2 = ws.reshape(NQ, E)
    out1 = _combine_sc(tab, idx4, w2).reshape(B, LQ, D)
    return _stage2(out1, W_out, b_out.reshape(1, D))


# speedup vs baseline: 102.1028x; 102.1028x over previous
import functools

import jax
import jax.numpy as jnp
import numpy as np
from jax import lax
from jax.experimental import pallas as pl
from jax.experimental.pallas import tpu as pltpu
from jax.experimental.pallas import tpu_sc as plsc

D = 256
NH = 8
NL = 4
NP = 4
HD = D // NH
NTAP = 4
_SHAPES = [(128, 128), (64, 64), (32, 32), (16, 16)]
LQ = sum(h * w for h, w in _SHAPES)
B = 2
BLK = 640
NBLK = LQ // BLK
E = NH * NL * NP * NTAP
ROWS = B * LQ * NH

_lane = np.arange(128)
_lane_l = (_lane // 4) % 4
_lane_h = _lane // 16
_W_l = np.array([_SHAPES[l][1] for l in _lane_l], np.float32)
_H_l = np.array([_SHAPES[l][0] for l in _lane_l], np.float32)
_lsi_np = np.cumsum([0] + [h * w for h, w in _SHAPES])[:-1]
_LSI_H = (_lsi_np[_lane_l] * NH + _lane_h).astype(np.int32)
_Sx = np.zeros((NL, 128), np.float32)
_Sx[_lane_l, _lane] = _W_l
_Sy = np.zeros((NL, 128), np.float32)
_Sy[_lane_l, _lane] = _H_l
_G = (_lane_h[:, None] == _lane_h[None, :]).astype(np.float32)


def _stage1_body(q_ref, v_ref, wv_ref, bv_ref, wa_ref, ba_ref, wox_ref,
                 woy_ref, bx_ref, by_ref, sx_ref, sy_ref, wl_ref, hl_ref,
                 lsih_ref, g_ref, rpx_ref, rpy_ref,
                 val_ref, idx_ref, w_ref):
    b = pl.program_id(0)
    q = q_ref[0]
    v = v_ref[0]
    val_ref[0] = jnp.dot(v, wv_ref[...], preferred_element_type=jnp.float32, precision=lax.Precision.HIGHEST) + bv_ref[...]
    logits = jnp.dot(q, wa_ref[...], preferred_element_type=jnp.float32, precision=lax.Precision.HIGHEST) + ba_ref[...]
    m = jnp.max(logits, axis=1, keepdims=True)
    e = jnp.exp(logits - m)
    den = jnp.dot(e, g_ref[...], preferred_element_type=jnp.float32, precision=lax.Precision.HIGHEST)
    att = e / den
    x = (jnp.dot(q, wox_ref[...], preferred_element_type=jnp.float32, precision=lax.Precision.HIGHEST)
         + jnp.dot(rpx_ref[0], sx_ref[...], preferred_element_type=jnp.float32, precision=lax.Precision.HIGHEST)
         + bx_ref[...])
    y = (jnp.dot(q, woy_ref[...], preferred_element_type=jnp.float32, precision=lax.Precision.HIGHEST)
         + jnp.dot(rpy_ref[0], sy_ref[...], preferred_element_type=jnp.float32, precision=lax.Precision.HIGHEST)
         + by_ref[...])
    wl = wl_ref[...]
    hl = hl_ref[...]
    x0 = jnp.floor(x)
    y0 = jnp.floor(y)
    fx1 = x - x0
    fx0 = 1.0 - fx1
    fy1 = y - y0
    fy0 = 1.0 - fy1
    base_b = b * (LQ * NH)
    lsih = lsih_ref[...]
    for t, (dx, dy) in enumerate(((0, 0), (1, 0), (0, 1), (1, 1))):
        xt = x0 + float(dx)
        yt = y0 + float(dy)
        valid = ((xt >= 0.0) & (xt <= wl - 1.0)
                 & (yt >= 0.0) & (yt <= hl - 1.0))
        xc = jnp.clip(xt, 0.0, wl - 1.0).astype(jnp.int32)
        yc = jnp.clip(yt, 0.0, hl - 1.0).astype(jnp.int32)
        spatial = yc * wl.astype(jnp.int32) + xc
        idx_ref[0, :, t * 128:(t + 1) * 128] = base_b + lsih + spatial * NH
        wt = (fx0 if dx == 0 else fx1) * (fy0 if dy == 0 else fy1)
        w_ref[0, :, t * 128:(t + 1) * 128] = att * wt * valid.astype(jnp.float32)


def _stage1(query, value, rpx, rpy, W_val, b_val, W_attn, b_attn,
            W_offx, W_offy, bx, by):
    consts = [
        jnp.asarray(_Sx), jnp.asarray(_Sy),
        jnp.asarray(_W_l).reshape(1, 128), jnp.asarray(_H_l).reshape(1, 128),
        jnp.asarray(_LSI_H).reshape(1, 128), jnp.asarray(_G),
    ]

    def whole(shape):
        return pl.BlockSpec(shape, lambda b, i: tuple(0 for _ in shape))

    return pl.pallas_call(
        _stage1_body,
        grid=(B, NBLK),
        in_specs=[
            pl.BlockSpec((1, BLK, D), lambda b, i: (b, i, 0)),
            pl.BlockSpec((1, BLK, D), lambda b, i: (b, i, 0)),
            whole((D, D)), whole((1, D)),
            whole((D, 128)), whole((1, 128)),
            whole((D, 128)), whole((D, 128)),
            whole((1, 128)), whole((1, 128)),
            whole((NL, 128)), whole((NL, 128)),
            whole((1, 128)), whole((1, 128)),
            whole((1, 128)), whole((128, 128)),
            pl.BlockSpec((1, BLK, NL), lambda b, i: (b, i, 0)),
            pl.BlockSpec((1, BLK, NL), lambda b, i: (b, i, 0)),
        ],
        out_specs=[
            pl.BlockSpec((1, BLK, D), lambda b, i: (b, i, 0)),
            pl.BlockSpec((1, BLK, E), lambda b, i: (b, i, 0)),
            pl.BlockSpec((1, BLK, E), lambda b, i: (b, i, 0)),
        ],
        out_shape=[
            jax.ShapeDtypeStruct((B, LQ, D), jnp.float32),
            jax.ShapeDtypeStruct((B, LQ, E), jnp.int32),
            jax.ShapeDtypeStruct((B, LQ, E), jnp.float32),
        ],
        compiler_params=pltpu.CompilerParams(
            dimension_semantics=("parallel", "parallel")),
    )(query, value, W_val, b_val, W_attn, b_attn, W_offx, W_offy, bx, by,
      *consts, rpx, rpy)


QC = 4
NQ = B * LQ
NGATH = E // 128


def _sc_body(tab_hbm, idx_hbm, w_hbm, out_hbm, idx_v, w_v, rows_v, out_v, sem):
    nc = 2
    wid = lax.axis_index("s") * nc + lax.axis_index("c")
    per_tile = NQ // 32
    iters = per_tile // QC
    qbase = wid * per_tile

    def it_body(i, carry):
        q0 = qbase + i * QC
        pltpu.sync_copy(idx_hbm.at[pl.ds(q0 * NGATH, QC * NGATH)], idx_v)
        pltpu.sync_copy(w_hbm.at[pl.ds(q0, QC)], w_v)
        handles = []
        for j in range(QC * NGATH):
            handles.append(pltpu.async_copy(
                tab_hbm.at[idx_v.at[j]],
                rows_v.at[pl.ds(j * 128, 128)], sem))
        for h in handles:
            h.wait()

        def qh_body(qh, c2):
            q = lax.shift_right_logical(qh, 3)
            hh = jnp.bitwise_and(qh, 7)

            def t_body(t, accs):
                a0, a1 = accs
                base = t * 128 + hh * 16
                wvec = w_v[q, pl.ds(base, 16)]
                for k in range(16):
                    wk = wvec[k]
                    r = q * E + base + k
                    a0 = a0 + wk * rows_v[r, 0:16]
                    a1 = a1 + wk * rows_v[r, 16:32]
                return (a0, a1)

            z = jnp.zeros((16,), jnp.float32)
            a0, a1 = lax.fori_loop(0, NTAP, t_body, (z, z))
            out_v[qh, 0:16] = a0
            out_v[qh, 16:32] = a1
            return c2

        lax.fori_loop(0, QC * NH, qh_body, 0)
        pltpu.sync_copy(out_v, out_hbm.at[pl.ds(q0 * NH, QC * NH)])
        return carry

    lax.fori_loop(0, iters, it_body, 0)


def _combine_sc(tab, idx4, w2):
    mesh = plsc.VectorSubcoreMesh(core_axis_name="c", subcore_axis_name="s")
    run = functools.partial(
        pl.kernel,
        mesh=mesh,
        out_type=jax.ShapeDtypeStruct((ROWS, HD), jnp.float32),
        scratch_types=[
            pltpu.VMEM((QC * NGATH, 128), jnp.int32),
            pltpu.VMEM((QC, E), jnp.float32),
            pltpu.VMEM((QC * E, HD), jnp.float32),
            pltpu.VMEM((QC * NH, HD), jnp.float32),
            pltpu.SemaphoreType.DMA,
        ],
        compiler_params=pltpu.CompilerParams(use_tc_tiling_on_sc=False),
    )(_sc_body)
    return run(tab, idx4, w2)


def _stage2_body(x_ref, w_ref, b_ref, o_ref):
    o_ref[0] = (jnp.dot(x_ref[0], w_ref[...], preferred_element_type=jnp.float32, precision=lax.Precision.HIGHEST)
                + b_ref[...])


def _stage2(x, W_out, b_out):
    return pl.pallas_call(
        _stage2_body,
        grid=(B, NBLK),
        in_specs=[
            pl.BlockSpec((1, BLK, D), lambda b, i: (b, i, 0)),
            pl.BlockSpec((D, D), lambda b, i: (0, 0)),
            pl.BlockSpec((1, D), lambda b, i: (0, 0)),
        ],
        out_specs=pl.BlockSpec((1, BLK, D), lambda b, i: (b, i, 0)),
        out_shape=jax.ShapeDtypeStruct((B, LQ, D), jnp.float32),
        compiler_params=pltpu.CompilerParams(
            dimension_semantics=("parallel", "parallel")),
    )(x, W_out, b_out)


_ATT_PERM = np.array([h * 16 + p * NL + l
                      for h in range(NH) for l in range(NL)
                      for p in range(NP)], np.int32)


def kernel(query, reference_points, value, spatial_shapes, level_start_index,
           W_off, b_off, W_attn, b_attn, W_val, b_val, W_out, b_out):
    W_attn = W_attn[:, _ATT_PERM]
    b_attn = b_attn[_ATT_PERM]
    rpx = reference_points[..., 0]
    rpy = reference_points[..., 1]
    W_offx = W_off[:, 0::2]
    W_offy = W_off[:, 1::2]
    bx = (b_off[0::2] - 0.5).reshape(1, 128)
    by = (b_off[1::2] - 0.5).reshape(1, 128)
    val, idxs, ws = _stage1(
        query, value, rpx, rpy, W_val, b_val.reshape(1, D),
        W_attn, b_attn.reshape(1, 128), W_offx, W_offy, bx, by)
    tab = val.reshape(ROWS, HD)
    idx4 = idxs.reshape(NQ * NGATH, 128)
    w2 = ws.reshape(NQ, E)
    out1 = _combine_sc(tab, idx4, w2).reshape(B, LQ, D)
    return _stage2(out1, W_out, b_out.reshape(1, D))


# speedup vs baseline: 132.5527x; 1.2982x over previous
import functools

import jax
import jax.numpy as jnp
import numpy as np
from jax import lax
from jax.experimental import pallas as pl
from jax.experimental.pallas import tpu as pltpu
from jax.experimental.pallas import tpu_sc as plsc

D = 256
NH = 8
NL = 4
NP = 4
HD = D // NH
NTAP = 4
_SHAPES = [(128, 128), (64, 64), (32, 32), (16, 16)]
LQ = sum(h * w for h, w in _SHAPES)
B = 2
BLK = 640
NBLK = LQ // BLK
E = NH * NL * NP * NTAP
ROWS = B * LQ * NH

_lane = np.arange(128)
_lane_l = (_lane // 4) % 4
_lane_h = _lane // 16
_W_l = np.array([_SHAPES[l][1] for l in _lane_l], np.float32)
_H_l = np.array([_SHAPES[l][0] for l in _lane_l], np.float32)
_lsi_np = np.cumsum([0] + [h * w for h, w in _SHAPES])[:-1]
_LSI_H = (_lsi_np[_lane_l] * NH + _lane_h).astype(np.int32)
_Sx = np.zeros((NL, 128), np.float32)
_Sx[_lane_l, _lane] = _W_l
_Sy = np.zeros((NL, 128), np.float32)
_Sy[_lane_l, _lane] = _H_l
_G = (_lane_h[:, None] == _lane_h[None, :]).astype(np.float32)


def _stage1_body(q_ref, v_ref, wv_ref, bv_ref, wa_ref, ba_ref, wox_ref,
                 woy_ref, bx_ref, by_ref, sx_ref, sy_ref, wl_ref, hl_ref,
                 lsih_ref, g_ref, rpx_ref, rpy_ref,
                 val_ref, idx_ref, w_ref):
    b = pl.program_id(0)
    q = q_ref[0]
    v = v_ref[0]
    val_ref[0] = jnp.dot(v, wv_ref[...], preferred_element_type=jnp.float32, precision=lax.Precision.HIGHEST) + bv_ref[...]
    logits = jnp.dot(q, wa_ref[...], preferred_element_type=jnp.float32, precision=lax.Precision.HIGHEST) + ba_ref[...]
    m = jnp.max(logits, axis=1, keepdims=True)
    e = jnp.exp(logits - m)
    den = jnp.dot(e, g_ref[...], preferred_element_type=jnp.float32, precision=lax.Precision.HIGHEST)
    att = e / den
    x = (jnp.dot(q, wox_ref[...], preferred_element_type=jnp.float32, precision=lax.Precision.HIGHEST)
         + jnp.dot(rpx_ref[0], sx_ref[...], preferred_element_type=jnp.float32, precision=lax.Precision.HIGHEST)
         + bx_ref[...])
    y = (jnp.dot(q, woy_ref[...], preferred_element_type=jnp.float32, precision=lax.Precision.HIGHEST)
         + jnp.dot(rpy_ref[0], sy_ref[...], preferred_element_type=jnp.float32, precision=lax.Precision.HIGHEST)
         + by_ref[...])
    wl = wl_ref[...]
    hl = hl_ref[...]
    x0 = jnp.floor(x)
    y0 = jnp.floor(y)
    fx1 = x - x0
    fx0 = 1.0 - fx1
    fy1 = y - y0
    fy0 = 1.0 - fy1
    base_b = b * (LQ * NH)
    lsih = lsih_ref[...]
    for t, (dx, dy) in enumerate(((0, 0), (1, 0), (0, 1), (1, 1))):
        xt = x0 + float(dx)
        yt = y0 + float(dy)
        valid = ((xt >= 0.0) & (xt <= wl - 1.0)
                 & (yt >= 0.0) & (yt <= hl - 1.0))
        xc = jnp.clip(xt, 0.0, wl - 1.0).astype(jnp.int32)
        yc = jnp.clip(yt, 0.0, hl - 1.0).astype(jnp.int32)
        spatial = yc * wl.astype(jnp.int32) + xc
        idx_ref[0, :, t * 128:(t + 1) * 128] = base_b + lsih + spatial * NH
        wt = (fx0 if dx == 0 else fx1) * (fy0 if dy == 0 else fy1)
        w_ref[0, :, t * 128:(t + 1) * 128] = att * wt * valid.astype(jnp.float32)


def _stage1(query, value, rpx, rpy, W_val, b_val, W_attn, b_attn,
            W_offx, W_offy, bx, by):
    consts = [
        jnp.asarray(_Sx), jnp.asarray(_Sy),
        jnp.asarray(_W_l).reshape(1, 128), jnp.asarray(_H_l).reshape(1, 128),
        jnp.asarray(_LSI_H).reshape(1, 128), jnp.asarray(_G),
    ]

    def whole(shape):
        return pl.BlockSpec(shape, lambda b, i: tuple(0 for _ in shape))

    return pl.pallas_call(
        _stage1_body,
        grid=(B, NBLK),
        in_specs=[
            pl.BlockSpec((1, BLK, D), lambda b, i: (b, i, 0)),
            pl.BlockSpec((1, BLK, D), lambda b, i: (b, i, 0)),
            whole((D, D)), whole((1, D)),
            whole((D, 128)), whole((1, 128)),
            whole((D, 128)), whole((D, 128)),
            whole((1, 128)), whole((1, 128)),
            whole((NL, 128)), whole((NL, 128)),
            whole((1, 128)), whole((1, 128)),
            whole((1, 128)), whole((128, 128)),
            pl.BlockSpec((1, BLK, NL), lambda b, i: (b, i, 0)),
            pl.BlockSpec((1, BLK, NL), lambda b, i: (b, i, 0)),
        ],
        out_specs=[
            pl.BlockSpec((1, BLK, D), lambda b, i: (b, i, 0)),
            pl.BlockSpec((1, BLK, E), lambda b, i: (b, i, 0)),
            pl.BlockSpec((1, BLK, E), lambda b, i: (b, i, 0)),
        ],
        out_shape=[
            jax.ShapeDtypeStruct((B, LQ, D), jnp.float32),
            jax.ShapeDtypeStruct((B, LQ, E), jnp.int32),
            jax.ShapeDtypeStruct((B, LQ, E), jnp.float32),
        ],
        compiler_params=pltpu.CompilerParams(
            dimension_semantics=("parallel", "parallel")),
    )(query, value, W_val, b_val, W_attn, b_attn, W_offx, W_offy, bx, by,
      *consts, rpx, rpy)


QC = 2
NQ = B * LQ
NGATH = E // 128


def _sc_body(tab_hbm, idx_hbm, w_hbm, out_hbm,
             idx_v, w_v, rows_v, out_v, gsem, osem):
    nc = 2
    wid = lax.axis_index("s") * nc + lax.axis_index("c")
    per_tile = NQ // 32
    iters = per_tile // QC
    qbase = wid * per_tile

    def fetch(q0, s):
        pltpu.sync_copy(idx_hbm.at[pl.ds(q0 * NGATH, QC * NGATH)], idx_v.at[s])
        pltpu.sync_copy(w_hbm.at[pl.ds(q0 * E, QC * E)], w_v.at[s])
        for j in range(QC * NGATH):
            pltpu.async_copy(
                tab_hbm.at[idx_v.at[s, j]],
                rows_v.at[s, pl.ds(j * 128, 128)], gsem)

    def drain(s):
        for j in range(QC * NGATH):
            pltpu.make_async_copy(
                tab_hbm.at[idx_v.at[s, j]],
                rows_v.at[s, pl.ds(j * 128, 128)], gsem).wait()

    def wait_out(s):
        pltpu.make_async_copy(
            out_v.at[s], out_hbm.at[pl.ds(0, QC * NH)], osem).wait()

    def compute(q0, s):
        def qh_body(qh, c2):
            rbase = (lax.shift_right_logical(qh, 3) * E
                     + jnp.bitwise_and(qh, 7) * 16)
            a0 = jnp.zeros((16,), jnp.float32)
            a1 = jnp.zeros((16,), jnp.float32)
            for t in range(NTAP):
                wvec = w_v[s, pl.ds(rbase + t * 128, 16)]
                for k in range(16):
                    wk = wvec[k]
                    r = rbase + t * 128 + k
                    a0 = a0 + wk * rows_v[s, r, 0:16]
                    a1 = a1 + wk * rows_v[s, r, 16:32]
            out_v[s, qh, 0:16] = a0
            out_v[s, qh, 16:32] = a1
            return c2

        lax.fori_loop(0, QC * NH, qh_body, 0)
        pltpu.async_copy(out_v.at[s], out_hbm.at[pl.ds(q0 * NH, QC * NH)], osem)

    npairs = iters // 2
    fetch(qbase, 0)

    def pair_body(p, carry):
        q0 = qbase + p * 2 * QC

        fetch(q0 + QC, 1)
        drain(0)

        @pl.when(p >= 1)
        def _():
            wait_out(0)

        compute(q0, 0)

        @pl.when(p + 1 < npairs)
        def _():
            fetch(q0 + 2 * QC, 0)

        drain(1)

        @pl.when(p >= 1)
        def _():
            wait_out(1)

        compute(q0 + QC, 1)
        return carry

    lax.fori_loop(0, npairs, pair_body, 0)
    wait_out(0)
    wait_out(1)


def _combine_sc(tab, idx4, w2):
    mesh = plsc.VectorSubcoreMesh(core_axis_name="c", subcore_axis_name="s")
    run = functools.partial(
        pl.kernel,
        mesh=mesh,
        out_type=jax.ShapeDtypeStruct((ROWS, HD), jnp.float32),
        scratch_types=[
            pltpu.VMEM((2, QC * NGATH, 128), jnp.int32),
            pltpu.VMEM((2, QC * E), jnp.float32),
            pltpu.VMEM((2, QC * E, HD), jnp.float32),
            pltpu.VMEM((2, QC * NH, HD), jnp.float32),
            pltpu.SemaphoreType.DMA,
            pltpu.SemaphoreType.DMA,
        ],
        compiler_params=pltpu.CompilerParams(use_tc_tiling_on_sc=False),
    )(_sc_body)
    return run(tab, idx4, w2)


def _stage2_body(x_ref, w_ref, b_ref, o_ref):
    o_ref[0] = (jnp.dot(x_ref[0], w_ref[...], preferred_element_type=jnp.float32, precision=lax.Precision.HIGHEST)
                + b_ref[...])


def _stage2(x, W_out, b_out):
    return pl.pallas_call(
        _stage2_body,
        grid=(B, NBLK),
        in_specs=[
            pl.BlockSpec((1, BLK, D), lambda b, i: (b, i, 0)),
            pl.BlockSpec((D, D), lambda b, i: (0, 0)),
            pl.BlockSpec((1, D), lambda b, i: (0, 0)),
        ],
        out_specs=pl.BlockSpec((1, BLK, D), lambda b, i: (b, i, 0)),
        out_shape=jax.ShapeDtypeStruct((B, LQ, D), jnp.float32),
        compiler_params=pltpu.CompilerParams(
            dimension_semantics=("parallel", "parallel")),
    )(x, W_out, b_out)


_ATT_PERM = np.array([h * 16 + p * NL + l
                      for h in range(NH) for l in range(NL)
                      for p in range(NP)], np.int32)


def kernel(query, reference_points, value, spatial_shapes, level_start_index,
           W_off, b_off, W_attn, b_attn, W_val, b_val, W_out, b_out):
    W_attn = W_attn[:, _ATT_PERM]
    b_attn = b_attn[_ATT_PERM]
    rpx = reference_points[..., 0]
    rpy = reference_points[..., 1]
    W_offx = W_off[:, 0::2]
    W_offy = W_off[:, 1::2]
    bx = (b_off[0::2] - 0.5).reshape(1, 128)
    by = (b_off[1::2] - 0.5).reshape(1, 128)
    val, idxs, ws = _stage1(
        query, value, rpx, rpy, W_val, b_val.reshape(1, D),
        W_attn, b_attn.reshape(1, 128), W_offx, W_offy, bx, by)
    tab = val.reshape(ROWS, HD)
    idx4 = idxs.reshape(NQ * NGATH, 128)
    w2 = ws.reshape(NQ * E)
    out1 = _combine_sc(tab, idx4, w2).reshape(B, LQ, D)
    return _stage2(out1, W_out, b_out.reshape(1, D))
